# Initial kernel scaffold; baseline (speedup 1.0000x reference)
import jax
import jax.numpy as jnp
from jax.experimental import pallas as pl


def kernel(exercise_id, weight_id, exercise_sequence, equipment_id, core, metric_type, exercise_table, weight_table, seq_table, equipment_table, core_table, metric_table, weight_fc_w, weight_fc_b, seq_fc_w, seq_fc_b, equipment_fc_w, equipment_fc_b, core_fc_w, core_fc_b, metric_fc_w, metric_fc_b):
    raise NotImplementedError("write your pallas kernel here")


# speedup vs baseline: 18.4578x; 18.4578x over previous
import functools

import jax
import jax.numpy as jnp
from jax import lax
from jax.experimental import pallas as pl
from jax.experimental.pallas import tpu as pltpu
from jax.experimental.pallas import tpu_sc as plsc

D_OUT = 8
NC, NS = 2, 16
NW = NC * NS
CHUNK = 1600


def _prep_body(wt_ref, st_ref, qt_ref, ct_ref, mt_ref,
               ww_ref, sw_ref, qw_ref, cw_ref, mw_ref,
               wb_ref, sb_ref, qb_ref, cb_ref, mb_ref,
               wt8_ref, qt8_ref, cms_ref):
    f32 = jnp.float32
    wt8_ref[...] = jnp.dot(wt_ref[...], ww_ref[...].T, preferred_element_type=f32)
    qt8_ref[...] = jnp.dot(qt_ref[...], qw_ref[...][:, :2].T, preferred_element_type=f32)
    st8 = jnp.dot(st_ref[...], sw_ref[...].T, preferred_element_type=f32)
    ct8 = jnp.dot(ct_ref[...], cw_ref[...].T, preferred_element_type=f32)
    mt8 = jnp.dot(mt_ref[...], mw_ref[...].T, preferred_element_type=f32)
    bias = wb_ref[...] + sb_ref[...] + qb_ref[...] + cb_ref[...] + mb_ref[...]
    cm = (ct8[:, None, :] + mt8[None, :, :]).reshape(8, D_OUT)
    cms = st8[:, None, :] + cm[None, :, :] + bias[None, None, :]
    cms_ref[...] = cms


def _sc_body(ex_t, wt8, qt8, cms, g_ex, g_w, g_q, g_s, g_c, g_m, out,
             i_ex, i_w, i_q, i_s, i_c, i_m, r_ex, r_w, r_q, r_cm,
             sem0, sem1, sem2, sem3, *, tokens_per_worker):
    wid = lax.axis_index("s") * NC + lax.axis_index("c")
    wbase = wid * tokens_per_worker
    nchunks = tokens_per_worker // CHUNK

    @pl.loop(0, nchunks)
    def _chunks(j):
        base = wbase + j * CHUNK
        sl = pl.ds(base, CHUNK)
        pltpu.sync_copy(g_ex.at[sl], i_ex)
        pltpu.sync_copy(g_w.at[sl], i_w)
        pltpu.sync_copy(g_q.at[sl], i_q)
        pltpu.sync_copy(g_s.at[sl], i_s)
        pltpu.sync_copy(g_c.at[sl], i_c)
        pltpu.sync_copy(g_m.at[sl], i_m)

        @pl.loop(0, CHUNK // 16)
        def _fold(k):
            ks = pl.ds(k * 16, 16)
            i_s[ks] = i_s[ks] * 8 + i_c[ks] * 4 + i_m[ks]

        cp0 = pltpu.make_async_copy(ex_t.at[i_ex], r_ex, sem0)
        cp1 = pltpu.make_async_copy(wt8.at[i_w], r_w, sem1)
        cp2 = pltpu.make_async_copy(qt8.at[i_q], r_q, sem2)
        cp3 = pltpu.make_async_copy(cms.at[i_s], r_cm, sem3)
        cp0.start(); cp1.start(); cp2.start(); cp3.start()
        cp0.wait(); cp1.wait(); cp2.wait(); cp3.wait()

        lane = lax.iota(jnp.int32, 16)
        rb = lane >> 3
        col = lane & 7

        @pl.loop(0, CHUNK * D_OUT // 16)
        def _add(k):
            row = rb + k * 2
            v = (plsc.load_gather(r_ex, [row, col])
                 + plsc.load_gather(r_w, [row, col])
                 + plsc.load_gather(r_q, [row, col])
                 + plsc.load_gather(r_cm, [row, col]))
            plsc.store_scatter(r_ex, [row, col], v)

        pltpu.sync_copy(r_ex, out.at[sl])


def kernel(exercise_id, weight_id, exercise_sequence, equipment_id, core, metric_type,
           exercise_table, weight_table, seq_table, equipment_table, core_table, metric_table,
           weight_fc_w, weight_fc_b, seq_fc_w, seq_fc_b, equipment_fc_w, equipment_fc_b,
           core_fc_w, core_fc_b, metric_fc_w, metric_fc_b):
    B, L = exercise_id.shape
    tok = B * L
    f32 = jnp.float32

    wt8, qt8, cms3 = pl.pallas_call(
        _prep_body,
        out_shape=[
            jax.ShapeDtypeStruct(weight_table.shape[:1] + (D_OUT,), f32),
            jax.ShapeDtypeStruct(equipment_table.shape[:1] + (D_OUT,), f32),
            jax.ShapeDtypeStruct((seq_table.shape[0], 8, D_OUT), f32),
        ],
    )(weight_table, seq_table, equipment_table, core_table, metric_table,
      weight_fc_w, seq_fc_w, equipment_fc_w, core_fc_w, metric_fc_w,
      weight_fc_b, seq_fc_b, equipment_fc_b, core_fc_b, metric_fc_b)
    cms = cms3.reshape(seq_table.shape[0] * 8, D_OUT)

    tokens_per_worker = tok // NW
    mesh = plsc.VectorSubcoreMesh(core_axis_name="c", subcore_axis_name="s",
                                  num_cores=NC, num_subcores=NS)
    sc = pl.kernel(
        functools.partial(_sc_body, tokens_per_worker=tokens_per_worker),
        out_type=jax.ShapeDtypeStruct((tok, D_OUT), f32),
        mesh=mesh,
        compiler_params=pltpu.CompilerParams(
            needs_layout_passes=False, use_tc_tiling_on_sc=False),
        scratch_types=[
            pltpu.VMEM((CHUNK,), jnp.int32),
            pltpu.VMEM((CHUNK,), jnp.int32),
            pltpu.VMEM((CHUNK,), jnp.int32),
            pltpu.VMEM((CHUNK,), jnp.int32),
            pltpu.VMEM((CHUNK,), jnp.int32),
            pltpu.VMEM((CHUNK,), jnp.int32),
            pltpu.VMEM((CHUNK, D_OUT), f32),
            pltpu.VMEM((CHUNK, D_OUT), f32),
            pltpu.VMEM((CHUNK, D_OUT), f32),
            pltpu.VMEM((CHUNK, D_OUT), f32),
            pltpu.SemaphoreType.DMA,
            pltpu.SemaphoreType.DMA,
            pltpu.SemaphoreType.DMA,
            pltpu.SemaphoreType.DMA,
        ],
    )
    out = sc(exercise_table, wt8, qt8, cms,
             exercise_id.reshape(-1), weight_id.reshape(-1),
             equipment_id.reshape(-1), exercise_sequence.reshape(-1),
             core.reshape(-1), metric_type.reshape(-1))
    return out.reshape(B, L, D_OUT)


# speedup vs baseline: 57.8233x; 3.1327x over previous
import functools

import jax
import jax.numpy as jnp
from jax import lax
from jax.experimental import pallas as pl
from jax.experimental.pallas import tpu as pltpu
from jax.experimental.pallas import tpu_sc as plsc

D_OUT = 8
NC, NS = 2, 16
NW = NC * NS
TILE_L, TILE_B = 8, 128
SLAB = TILE_L * TILE_B


def _prep_body(wt_ref, st_ref, qt_ref, ct_ref, mt_ref,
               ww_ref, sw_ref, qw_ref, cw_ref, mw_ref,
               wb_ref, sb_ref, qb_ref, cb_ref, mb_ref,
               wt8_ref, qt8_ref, cms_ref):
    f32 = jnp.float32
    wt8_ref[...] = jnp.dot(wt_ref[...], ww_ref[...].T, preferred_element_type=f32)
    qt8_ref[...] = jnp.dot(qt_ref[...], qw_ref[...][:, :2].T, preferred_element_type=f32)
    st8 = jnp.dot(st_ref[...], sw_ref[...].T, preferred_element_type=f32)
    ct8 = jnp.dot(ct_ref[...], cw_ref[...].T, preferred_element_type=f32)
    mt8 = jnp.dot(mt_ref[...], mw_ref[...].T, preferred_element_type=f32)
    bias = wb_ref[...] + sb_ref[...] + qb_ref[...] + cb_ref[...] + mb_ref[...]
    cm = (ct8[:, None, :] + mt8[None, :, :]).reshape(8, D_OUT)
    cms = st8[:, None, :] + cm[None, :, :] + bias[None, None, :]
    cms_ref[...] = cms


def _sc_body(ex_t, wt8, qt8, cms, g_ex, g_w, g_q, g_s, g_c, g_m, out,
             idx, rows, acc, sem_i, sem_g, sem_o, *, n_lb):
    w = lax.axis_index("s") * NC + lax.axis_index("c")
    lane = lax.iota(jnp.int32, 16)
    rowbase = lane >> 3
    dcol = lane & 7
    colbase = dcol * TILE_B + rowbase

    def load_fold_gather(lb, s):
        gi = idx[s]
        cps = [pltpu.make_async_copy(g.at[lb, w], gi[i], sem_i[s])
               for i, g in enumerate((g_ex, g_w, g_q, g_s, g_c, g_m))]
        for cp in cps:
            cp.start()
        for cp in cps:
            cp.wait()

        @pl.loop(0, SLAB // 16, unroll=8)
        def _fold(k):
            ks = pl.ds(k * 16, 16)
            gi[3][ks] = gi[3][ks] * 8 + gi[4][ks] * 4 + gi[5][ks]

        for cp in _gather_descs(lb, s):
            cp.start()

    def _gather_descs(lb, s):
        gi, rw = idx[s], rows[s]
        return [pltpu.make_async_copy(ex_t.at[gi[0]], rw[0], sem_g[s]),
                pltpu.make_async_copy(wt8.at[gi[1]], rw[1], sem_g[s]),
                pltpu.make_async_copy(qt8.at[gi[2]], rw[2], sem_g[s]),
                pltpu.make_async_copy(cms.at[gi[3]], rw[3], sem_g[s])]

    def do_round(r, s):
        @pl.when(r + 1 < n_lb)
        def _prefetch():
            load_fold_gather(r + 1, 1 - s)

        for cp in _gather_descs(r, s):
            cp.wait()

        @pl.when(r >= 2)
        def _drain_out():
            pltpu.make_async_copy(acc[s], out.at[pl.ds((r - 2) * TILE_L, TILE_L), w],
                                  sem_o[s]).wait()

        r0, r1, r2, r3 = rows[s]
        a = acc[s]

        @pl.loop(0, SLAB * D_OUT // 16, unroll=8)
        def _add(k):
            trow = rowbase + 2 * k
            v = (plsc.load_gather(r0, [trow, dcol])
                 + plsc.load_gather(r1, [trow, dcol])
                 + plsc.load_gather(r2, [trow, dcol])
                 + plsc.load_gather(r3, [trow, dcol]))
            li = jnp.broadcast_to(k >> 6, (16,)).astype(jnp.int32)
            col = colbase + 2 * (k & 63)
            plsc.store_scatter(a, [li, col], v)

        pltpu.make_async_copy(a, out.at[pl.ds(r * TILE_L, TILE_L), w],
                              sem_o[s]).start()

    load_fold_gather(0, 0)

    @pl.loop(0, (n_lb + 1) // 2)
    def _round_pair(h):
        for sub in (0, 1):
            r = h * 2 + sub

            @pl.when(r < n_lb)
            def _do(r=r, sub=sub):
                do_round(r, sub)

    for s, r in ((n_lb % 2, n_lb - 2), ((n_lb - 1) % 2, n_lb - 1)):
        pltpu.make_async_copy(acc[s], out.at[pl.ds(r * TILE_L, TILE_L), w],
                              sem_o[s]).wait()


def kernel(exercise_id, weight_id, exercise_sequence, equipment_id, core, metric_type,
           exercise_table, weight_table, seq_table, equipment_table, core_table, metric_table,
           weight_fc_w, weight_fc_b, seq_fc_w, seq_fc_b, equipment_fc_w, equipment_fc_b,
           core_fc_w, core_fc_b, metric_fc_w, metric_fc_b):
    B, L = exercise_id.shape
    f32 = jnp.float32
    n_lb, n_bb = L // TILE_L, B // TILE_B

    wt8, qt8, cms3 = pl.pallas_call(
        _prep_body,
        out_shape=[
            jax.ShapeDtypeStruct(weight_table.shape[:1] + (D_OUT,), f32),
            jax.ShapeDtypeStruct(equipment_table.shape[:1] + (D_OUT,), f32),
            jax.ShapeDtypeStruct((seq_table.shape[0], 8, D_OUT), f32),
        ],
    )(weight_table, seq_table, equipment_table, core_table, metric_table,
      weight_fc_w, seq_fc_w, equipment_fc_w, core_fc_w, metric_fc_w,
      weight_fc_b, seq_fc_b, equipment_fc_b, core_fc_b, metric_fc_b)
    cms = cms3.reshape(seq_table.shape[0] * 8, D_OUT)

    def tiled_view(a):
        return (a.T.reshape(n_lb, TILE_L, n_bb, TILE_B)
                .transpose(0, 2, 1, 3).reshape(n_lb, n_bb, SLAB))

    mesh = plsc.VectorSubcoreMesh(core_axis_name="c", subcore_axis_name="s",
                                  num_cores=NC, num_subcores=NS)
    sc = pl.kernel(
        functools.partial(_sc_body, n_lb=n_lb),
        out_type=jax.ShapeDtypeStruct((L, n_bb, SLAB), f32),
        mesh=mesh,
        scratch_types=[
            [[pltpu.VMEM((SLAB,), jnp.int32) for _ in range(6)] for _ in range(2)],
            [[pltpu.VMEM((SLAB, D_OUT), f32) for _ in range(4)] for _ in range(2)],
            [pltpu.VMEM((TILE_L, SLAB), f32) for _ in range(2)],
            [pltpu.SemaphoreType.DMA for _ in range(2)],
            [pltpu.SemaphoreType.DMA for _ in range(2)],
            [pltpu.SemaphoreType.DMA for _ in range(2)],
        ],
        compiler_params=pltpu.CompilerParams(
            needs_layout_passes=False, use_tc_tiling_on_sc=False),
    )
    out = sc(exercise_table, wt8, qt8, cms,
             tiled_view(exercise_id), tiled_view(weight_id),
             tiled_view(equipment_id), tiled_view(exercise_sequence),
             tiled_view(core), tiled_view(metric_type))
    return (out.reshape(L, n_bb, D_OUT, TILE_B).transpose(1, 3, 0, 2)
            .reshape(B, L, D_OUT))
